# baseline (speedup 1.0000x reference)
import functools

import jax
import jax.numpy as jnp
from jax import lax
from jax.experimental import pallas as pl
from jax.experimental.pallas import tpu as pltpu
from jax.experimental.pallas import tpu_sc as plsc

N = 10000
E = 320000
NP = 10240
NCORE = 2
NSUB = 16
NW = NCORE * NSUB
EB = 128
NB = 84
EP = NW * NB * EB
RPT = NP // NSUB

_mesh = plsc.VectorSubcoreMesh(
    core_axis_name="c", subcore_axis_name="s", num_cores=NCORE,
    num_subcores=NSUB)

_f32 = jnp.float32
_i32 = jnp.int32

_sc_params = pltpu.CompilerParams(use_tc_tiling_on_sc=False,
                                  needs_layout_passes=False)


def _fill_rows(ref, nrows, ncols, value):
    def body(r, carry):
        for c in range(ncols // 16):
            ref[r, pl.ds(c * 16, 16)] = jnp.full((16,), value, _f32)
        return carry
    lax.fori_loop(0, nrows, body, 0, unroll=4)


def _zero_spmem(acc_sh, zbuf, sid, ncols):
    _fill_rows(zbuf, EB, ncols, 0.0)
    for i in range(RPT // EB):
        pltpu.sync_copy(zbuf, acc_sh.at[pl.ds(sid * RPT + i * EB, EB)])


def _copy_out(acc_sh, out0, out1, cid, sid):
    ds = pl.ds(sid * RPT, RPT)
    @pl.when(cid == 0)
    def _():
        pltpu.sync_copy(acc_sh.at[ds], out0.at[ds])
    @pl.when(cid == 1)
    def _():
        pltpu.sync_copy(acc_sh.at[ds], out1.at[ds])


@functools.partial(
    pl.kernel, mesh=_mesh,
    out_type=(jax.ShapeDtypeStruct((NP, 16), _f32),
              jax.ShapeDtypeStruct((NP, 16), _f32)),
    scratch_types=[
        pltpu.VMEM((NB, EB), _i32),
        pltpu.VMEM((EB, 16), _f32),
        pltpu.SemaphoreType.DMA,
        pltpu.VMEM_SHARED((NP, 16), _f32),
    ],
    compiler_params=_sc_params,
)
def _sc_degree(dst_hbm, out0, out1, dst_v, buf, sem, acc_sh):
    cid = lax.axis_index("c")
    sid = lax.axis_index("s")
    wid = sid * NCORE + cid
    _zero_spmem(acc_sh, buf, sid, 16)
    plsc.subcore_barrier()
    pltpu.sync_copy(dst_hbm.at[wid], dst_v)
    _fill_rows(buf, EB, 16, 1.0)

    def body(j, carry):
        pltpu.sync_copy(buf, acc_sh.at[dst_v.at[j]], add=True)
        return carry
    lax.fori_loop(0, NB, body, 0)
    plsc.subcore_barrier()
    _copy_out(acc_sh, out0, out1, cid, sid)


def _make_sc_agg(F):
    @functools.partial(
        pl.kernel, mesh=_mesh,
        out_type=(jax.ShapeDtypeStruct((NP, F), _f32),
                  jax.ShapeDtypeStruct((NP, F), _f32)),
        scratch_types=[
            pltpu.VMEM((NB, EB), _i32),
            pltpu.VMEM((NB, EB), _i32),
            pltpu.VMEM((EB, F), _f32),
            pltpu.VMEM((EB, F), _f32),
            pltpu.SemaphoreType.DMA,
            pltpu.SemaphoreType.DMA,
            pltpu.VMEM_SHARED((NP, F), _f32),
            pltpu.VMEM_SHARED((NP, F), _f32),
        ],
        name=f"sc_gcn_agg_{F}",
        compiler_params=_sc_params,
    )
    def k(src_hbm, dst_hbm, g_hbm, out0, out1,
          src_v, dst_v, b0, b1, s0, s1, acc_sh, g_sh):
        cid = lax.axis_index("c")
        sid = lax.axis_index("s")
        wid = sid * NCORE + cid
        rs = pl.ds(sid * RPT, RPT)
        pltpu.sync_copy(g_hbm.at[rs], g_sh.at[rs])
        _zero_spmem(acc_sh, b0, sid, F)
        plsc.subcore_barrier()
        pltpu.sync_copy(src_hbm.at[wid], src_v)
        pltpu.sync_copy(dst_hbm.at[wid], dst_v)

        pltpu.async_copy(g_sh.at[src_v.at[0]], b0, s0)
        pltpu.async_copy(g_sh.at[src_v.at[1]], b1, s1)

        def body(jj, carry):
            j = jj * 2
            pltpu.make_async_copy(g_sh.at[src_v.at[0]], b0, s0).wait()
            pltpu.sync_copy(b0, acc_sh.at[dst_v.at[j]], add=True)
            @pl.when(j + 2 < NB)
            def _():
                pltpu.async_copy(g_sh.at[src_v.at[j + 2]], b0, s0)
            pltpu.make_async_copy(g_sh.at[src_v.at[1]], b1, s1).wait()
            pltpu.sync_copy(b1, acc_sh.at[dst_v.at[j + 1]], add=True)
            @pl.when(j + 3 < NB)
            def _():
                pltpu.async_copy(g_sh.at[src_v.at[j + 3]], b1, s1)
            return carry
        lax.fori_loop(0, NB // 2, body, 0)
        plsc.subcore_barrier()
        _copy_out(acc_sh, out0, out1, cid, sid)
    return k


_sc_agg64 = _make_sc_agg(64)
_sc_agg32 = _make_sc_agg(32)


GF = 48


@functools.partial(
    pl.kernel, mesh=_mesh,
    out_type=(jax.ShapeDtypeStruct((NP, GF), _f32),
              jax.ShapeDtypeStruct((NP, GF), _f32)),
    scratch_types=[
        pltpu.VMEM((NB, EB), _i32),
        pltpu.VMEM((NB, EB), _i32),
        pltpu.VMEM((NP,), _f32),
        pltpu.VMEM((NP,), _f32),
        pltpu.VMEM((16,), _f32),
        pltpu.VMEM((EB, 32), _f32),
        pltpu.VMEM((EB, 32), _f32),
        pltpu.VMEM((EB, 32), _f32),
        pltpu.VMEM((EB, 32), _f32),
        pltpu.VMEM((EB, GF), _f32),
        pltpu.VMEM((EB, GF), _f32),
        pltpu.SemaphoreType.DMA,
        pltpu.SemaphoreType.DMA,
        pltpu.SemaphoreType.DMA,
        pltpu.SemaphoreType.DMA,
        pltpu.VMEM_SHARED((NP, GF), _f32),
        pltpu.VMEM_SHARED((NP, 32), _f32),
    ],
    name="sc_gat_agg",
    compiler_params=_sc_params,
)
def _sc_gat(src_hbm, dst_hbm, hg_hbm, als_hbm, ald_hbm, m_hbm,
            out0, out1, src_v, dst_v, als_v, ald_v, m_v,
            b0, b1, b2, b3, obuf0, obuf1, sg0, sg1, ss0, ss1, acc_sh, hg_sh):
    cid = lax.axis_index("c")
    sid = lax.axis_index("s")
    wid = sid * NCORE + cid
    bufs = (b0, b1, b2, b3)
    obufs = (obuf0, obuf1)
    gsems = (sg0, sg1)
    ssems = (ss0, ss1)
    rs = pl.ds(sid * RPT, RPT)
    pltpu.sync_copy(hg_hbm.at[rs], hg_sh.at[rs])
    _zero_spmem(acc_sh, obuf0, sid, GF)
    plsc.subcore_barrier()
    pltpu.sync_copy(src_hbm.at[wid], src_v)
    pltpu.sync_copy(dst_hbm.at[wid], dst_v)
    pltpu.sync_copy(als_hbm, als_v)
    pltpu.sync_copy(ald_hbm, ald_v)
    pltpu.sync_copy(m_hbm, m_v)
    mvec = m_v[pl.ds(0, 16)]
    for ob in obufs:
        def zbody(r, carry):
            ob[r, pl.ds(32, 16)] = jnp.zeros((16,), _f32)
            return carry
        lax.fori_loop(0, EB, zbody, 0, unroll=4)

    for j in range(2):
        pltpu.async_copy(hg_sh.at[src_v.at[j]], bufs[j], gsems[j])

    def process(j, gbuf, ob):
        for g in range(EB // 16):
            sidx = src_v[j, pl.ds(g * 16, 16)]
            didx = dst_v[j, pl.ds(g * 16, 16)]
            el = plsc.load_gather(als_v, [sidx]) + plsc.load_gather(
                ald_v, [didx])
            e = jnp.where(el > 0, el, el * jnp.float32(0.2))
            p = jnp.exp(e - mvec)
            rows = lax.iota(_i32, 16) + g * 16
            cols = jnp.full((16,), 32, _i32)
            plsc.store_scatter(ob, [rows, cols], p)
            for t in range(16):
                r = g * 16 + t
                pb = p[jnp.full((16,), t, _i32)]
                ob[r, pl.ds(0, 16)] = gbuf[r, pl.ds(0, 16)] * pb
                ob[r, pl.ds(16, 16)] = gbuf[r, pl.ds(16, 16)] * pb

    def body(ii, carry):
        for q in range(4):
            j = ii * 4 + q
            bg = bufs[q]
            bn = bufs[(q + 2) % 4]
            ob = obufs[q % 2]
            @pl.when(j >= 2)
            def _():
                pltpu.make_async_copy(ob, acc_sh.at[dst_v.at[0]],
                                      ssems[q % 2]).wait()
            pltpu.make_async_copy(hg_sh.at[src_v.at[0]], bg,
                                  gsems[q % 2]).wait()
            @pl.when(j + 2 < NB)
            def _():
                pltpu.async_copy(hg_sh.at[src_v.at[j + 2]], bn,
                                 gsems[q % 2])
            process(j, bg, ob)
            pltpu.async_copy(ob, acc_sh.at[dst_v.at[j]], ssems[q % 2],
                             add=True)
        return carry
    lax.fori_loop(0, NB // 4, body, 0)
    for q in range(2):
        pltpu.make_async_copy(obufs[q], acc_sh.at[dst_v.at[0]],
                              ssems[q]).wait()
    plsc.subcore_barrier()
    _copy_out(acc_sh, out0, out1, cid, sid)


def _tc_call(body, out_shapes, *args):
    return pl.pallas_call(body, out_shape=out_shapes)(*args)


def _tcA1_body(x, w1, h1_o):
    h1_o[:, :] = jnp.dot(x[:, :], w1[:, :], preferred_element_type=_f32)


def _tcA2_body(d0, d1, h1, dinv_o, g1_o):
    deg = d0[:, 0:1] + d1[:, 0:1]
    dinv = jnp.where(deg > 0, lax.rsqrt(deg), jnp.float32(0.0))
    dinv_o[:, :] = dinv
    g1_o[:, :] = h1[:, :] * dinv


def _tcB_body(p0, p1, dinv, b, w, g_o):
    dv = dinv[:, :]
    xk = jax.nn.relu(dv * (p0[:, :] + p1[:, :]) + b[:, :])
    g_o[:, :] = jnp.dot(xk, w[:, :], preferred_element_type=_f32) * dv


def _tcD_body(p0, p1, dinv, b, wg, ats, atd, hg_o, als_o, ald_o, m_o):
    dv = dinv[:, :]
    x4 = jax.nn.relu(dv * (p0[:, :] + p1[:, :]) + b[:, :])
    hg = jnp.dot(x4, wg[:, :], preferred_element_type=_f32)
    hg_o[:, :] = hg
    als = jnp.sum(hg * ats[:, :], axis=1, keepdims=True)
    ald = jnp.sum(hg * atd[:, :], axis=1, keepdims=True)
    als_o[:, :] = als
    ald_o[:, :] = ald
    m = jax.nn.relu(jnp.max(als) + jnp.max(ald))
    m_o[:, :] = jnp.full((1, 16), m, _f32)


def _lstm_block(x, wt, bi, bh):
    g = jnp.dot(x, wt[:, :], preferred_element_type=_f32) + bi[:, :] + bh[:, :]
    i = g[:, 0:128]
    c = g[:, 256:384]
    o = g[:, 384:512]
    cs = jax.nn.sigmoid(i) * jnp.tanh(c)
    return jax.nn.sigmoid(o) * jnp.tanh(cs)


def _tcF_body(g0, g1, bg, w0f, bi0f, bh0f, w0b, bi0b, bh0b,
              w1f, bi1f, bh1f, w1b, bi1b, bh1b,
              gam, bet, mu, var, wf, bf, y_o):
    s = g0[:, 0:32] + g1[:, 0:32]
    den = g0[:, 32:33] + g1[:, 32:33]
    den = jnp.where(den > 0, den, jnp.float32(1.0))
    x5 = jax.nn.relu(s / den + bg[:, :])
    hf = _lstm_block(x5, w0f, bi0f, bh0f)
    hb = _lstm_block(x5, w0b, bi0b, bh0b)
    h1 = jnp.concatenate([hf, hb], axis=1)
    h1f = _lstm_block(h1, w1f, bi1f, bh1f)
    h1b = _lstm_block(h1, w1b, bi1b, bh1b)
    out = jnp.concatenate([h1f, h1b], axis=1)
    out = (out - mu[:, :]) * lax.rsqrt(var[:, :] + 1e-5) * gam[:, :] + bet[:, :]
    y_o[:, :] = jnp.dot(out, wf[:, :], preferred_element_type=_f32) + bf[:, :]


_BLK = 1024


def _tc_head(g0, g1, bg, w0f, bi0f, bh0f, w0b, bi0b, bh0b,
             w1f, bi1f, bh1f, w1b, bi1b, bh1b, gam, bet, mu, var, wf, bf):
    rep2 = lambda shape: pl.BlockSpec(shape, lambda i: (0, 0))
    blk = lambda shape: pl.BlockSpec(shape, lambda i: (i, 0))
    return pl.pallas_call(
        _tcF_body,
        grid=(NP // _BLK,),
        in_specs=[
            blk((_BLK, GF)), blk((_BLK, GF)), rep2((1, 32)),
            rep2((32, 512)), rep2((1, 512)), rep2((1, 512)),
            rep2((32, 512)), rep2((1, 512)), rep2((1, 512)),
            rep2((256, 512)), rep2((1, 512)), rep2((1, 512)),
            rep2((256, 512)), rep2((1, 512)), rep2((1, 512)),
            rep2((1, 256)), rep2((1, 256)), rep2((1, 256)), rep2((1, 256)),
            rep2((256, 64)), rep2((1, 64)),
        ],
        out_specs=blk((_BLK, 64)),
        out_shape=jax.ShapeDtypeStruct((NP, 64), _f32),
    )(g0, g1, bg, w0f, bi0f, bh0f, w0b, bi0b, bh0b,
      w1f, bi1f, bh1f, w1b, bi1b, bh1b, gam, bet, mu, var, wf, bf)


def kernel(x, edge_index, W1, b1, W2, b2, W3, b3, Wg, att_src, att_dst, bg,
           Wih0f, bih0f, bhh0f, Wih0b, bih0b, bhh0b,
           Wih1f, bih1f, bhh1f, Wih1b, bih1b, bhh1b,
           bn_gamma, bn_beta, bn_mean, bn_var, Wf, bf):
    loop = jnp.arange(N, dtype=edge_index.dtype)
    pad = jnp.full((EP - E - N,), N, edge_index.dtype)
    src3 = jnp.concatenate([edge_index[0], loop, pad]).reshape(NW, NB, EB)
    dst3 = jnp.concatenate([edge_index[1], loop, pad]).reshape(NW, NB, EB)
    x_p = jnp.pad(x, ((0, NP - N), (0, 0)))

    r2 = lambda a: a.reshape(1, -1)

    d0, d1 = _sc_degree(dst3)
    H1 = _tc_call(_tcA1_body, jax.ShapeDtypeStruct((NP, 64), _f32), x_p, W1)
    dinv, G1 = _tc_call(
        _tcA2_body,
        (jax.ShapeDtypeStruct((NP, 1), _f32),
         jax.ShapeDtypeStruct((NP, 64), _f32)),
        d0, d1, H1)

    a0, a1 = _sc_agg64(src3, dst3, G1)
    G2 = _tc_call(_tcB_body, jax.ShapeDtypeStruct((NP, 32), _f32),
                  a0, a1, dinv, r2(b1), W2)
    a0, a1 = _sc_agg32(src3, dst3, G2)
    G3 = _tc_call(_tcB_body, jax.ShapeDtypeStruct((NP, 32), _f32),
                  a0, a1, dinv, r2(b2), W3)
    a0, a1 = _sc_agg32(src3, dst3, G3)
    Hg, als, ald, m16 = _tc_call(
        _tcD_body,
        (jax.ShapeDtypeStruct((NP, 32), _f32),
         jax.ShapeDtypeStruct((NP, 1), _f32),
         jax.ShapeDtypeStruct((NP, 1), _f32),
         jax.ShapeDtypeStruct((1, 16), _f32)),
        a0, a1, dinv, r2(b3), Wg, r2(att_src), r2(att_dst))

    g0, g1 = _sc_gat(src3, dst3, Hg, als.reshape(NP), ald.reshape(NP),
                     m16.reshape(16))

    y = _tc_head(g0, g1, r2(bg),
                 Wih0f.T, r2(bih0f), r2(bhh0f),
                 Wih0b.T, r2(bih0b), r2(bhh0b),
                 Wih1f.T, r2(bih1f), r2(bhh1f),
                 Wih1b.T, r2(bih1b), r2(bhh1b),
                 r2(bn_gamma), r2(bn_beta), r2(bn_mean), r2(bn_var),
                 Wf, r2(bf))
    return y[:N]


# speedup vs baseline: 1.0288x; 1.0288x over previous
import functools

import jax
import jax.numpy as jnp
from jax import lax
from jax.experimental import pallas as pl
from jax.experimental.pallas import tpu as pltpu
from jax.experimental.pallas import tpu_sc as plsc

N = 10000
E = 320000
NP = 10240
NCORE = 2
NSUB = 16
NW = NCORE * NSUB
EB = 128
NB = 80
EP = NW * NB * EB
RPT = NP // NSUB

_mesh = plsc.VectorSubcoreMesh(
    core_axis_name="c", subcore_axis_name="s", num_cores=NCORE,
    num_subcores=NSUB)

_f32 = jnp.float32
_i32 = jnp.int32

_sc_params = pltpu.CompilerParams(use_tc_tiling_on_sc=False,
                                  needs_layout_passes=False)


def _fill_rows(ref, nrows, ncols, value):
    def body(r, carry):
        for c in range(ncols // 16):
            ref[r, pl.ds(c * 16, 16)] = jnp.full((16,), value, _f32)
        return carry
    lax.fori_loop(0, nrows, body, 0, unroll=4)


def _zero_spmem(acc_sh, zbuf, sid, ncols):
    _fill_rows(zbuf, EB, ncols, 0.0)
    for i in range(RPT // EB):
        pltpu.sync_copy(zbuf, acc_sh.at[pl.ds(sid * RPT + i * EB, EB)])


def _copy_out(acc_sh, out0, out1, cid, sid):
    ds = pl.ds(sid * RPT, RPT)
    @pl.when(cid == 0)
    def _():
        pltpu.sync_copy(acc_sh.at[ds], out0.at[ds])
    @pl.when(cid == 1)
    def _():
        pltpu.sync_copy(acc_sh.at[ds], out1.at[ds])


@functools.partial(
    pl.kernel, mesh=_mesh,
    out_type=(jax.ShapeDtypeStruct((NP, 16), _f32),
              jax.ShapeDtypeStruct((NP, 16), _f32)),
    scratch_types=[
        pltpu.VMEM((NB, EB), _i32),
        pltpu.VMEM((EB, 16), _f32),
        pltpu.SemaphoreType.DMA,
        pltpu.VMEM_SHARED((NP, 16), _f32),
    ],
    compiler_params=_sc_params,
)
def _sc_degree(dst_hbm, out0, out1, dst_v, buf, sem, acc_sh):
    cid = lax.axis_index("c")
    sid = lax.axis_index("s")
    wid = sid * NCORE + cid
    _zero_spmem(acc_sh, buf, sid, 16)
    plsc.subcore_barrier()
    pltpu.sync_copy(dst_hbm.at[wid], dst_v)
    _fill_rows(buf, EB, 16, 1.0)

    def body(j, carry):
        pltpu.sync_copy(buf, acc_sh.at[dst_v.at[j]], add=True)
        return carry
    lax.fori_loop(0, NB, body, 0)
    plsc.subcore_barrier()
    _copy_out(acc_sh, out0, out1, cid, sid)


def _make_sc_agg(F):
    @functools.partial(
        pl.kernel, mesh=_mesh,
        out_type=(jax.ShapeDtypeStruct((NP, F), _f32),
                  jax.ShapeDtypeStruct((NP, F), _f32)),
        scratch_types=[
            pltpu.VMEM((NB, EB), _i32),
            pltpu.VMEM((NB, EB), _i32),
            pltpu.VMEM((EB, F), _f32),
            pltpu.VMEM((EB, F), _f32),
            pltpu.SemaphoreType.DMA,
            pltpu.SemaphoreType.DMA,
            pltpu.VMEM_SHARED((NP, F), _f32),
            pltpu.VMEM_SHARED((NP, F), _f32),
        ],
        name=f"sc_gcn_agg_{F}",
        compiler_params=_sc_params,
    )
    def k(src_hbm, dst_hbm, g_hbm, out0, out1,
          src_v, dst_v, b0, b1, s0, s1, acc_sh, g_sh):
        cid = lax.axis_index("c")
        sid = lax.axis_index("s")
        wid = sid * NCORE + cid
        rs = pl.ds(sid * RPT, RPT)
        pltpu.sync_copy(g_hbm.at[rs], g_sh.at[rs])
        _zero_spmem(acc_sh, b0, sid, F)
        plsc.subcore_barrier()
        pltpu.sync_copy(src_hbm.at[wid], src_v)
        pltpu.sync_copy(dst_hbm.at[wid], dst_v)

        pltpu.async_copy(g_sh.at[src_v.at[0]], b0, s0)
        pltpu.async_copy(g_sh.at[src_v.at[1]], b1, s1)

        def body(jj, carry):
            j = jj * 2
            pltpu.make_async_copy(g_sh.at[src_v.at[0]], b0, s0).wait()
            pltpu.sync_copy(b0, acc_sh.at[dst_v.at[j]], add=True)
            @pl.when(j + 2 < NB)
            def _():
                pltpu.async_copy(g_sh.at[src_v.at[j + 2]], b0, s0)
            pltpu.make_async_copy(g_sh.at[src_v.at[1]], b1, s1).wait()
            pltpu.sync_copy(b1, acc_sh.at[dst_v.at[j + 1]], add=True)
            @pl.when(j + 3 < NB)
            def _():
                pltpu.async_copy(g_sh.at[src_v.at[j + 3]], b1, s1)
            return carry
        lax.fori_loop(0, NB // 2, body, 0)
        plsc.subcore_barrier()
        _copy_out(acc_sh, out0, out1, cid, sid)
    return k


_sc_agg64 = _make_sc_agg(64)
_sc_agg32 = _make_sc_agg(32)


GF = 48


@functools.partial(
    pl.kernel, mesh=_mesh,
    out_type=(jax.ShapeDtypeStruct((NP, GF), _f32),
              jax.ShapeDtypeStruct((NP, GF), _f32)),
    scratch_types=[
        pltpu.VMEM((NB, EB), _i32),
        pltpu.VMEM((NB, EB), _i32),
        pltpu.VMEM((NP,), _f32),
        pltpu.VMEM((NP,), _f32),
        pltpu.VMEM((16,), _f32),
        pltpu.VMEM((EB, 32), _f32),
        pltpu.VMEM((EB, 32), _f32),
        pltpu.VMEM((EB, 32), _f32),
        pltpu.VMEM((EB, 32), _f32),
        pltpu.VMEM((EB, GF), _f32),
        pltpu.VMEM((EB, GF), _f32),
        pltpu.SemaphoreType.DMA,
        pltpu.SemaphoreType.DMA,
        pltpu.SemaphoreType.DMA,
        pltpu.SemaphoreType.DMA,
        pltpu.VMEM_SHARED((NP, GF), _f32),
        pltpu.VMEM_SHARED((NP, 32), _f32),
    ],
    name="sc_gat_agg",
    compiler_params=_sc_params,
)
def _sc_gat(src_hbm, dst_hbm, hg_hbm, als_hbm, ald_hbm, m_hbm,
            out0, out1, src_v, dst_v, als_v, ald_v, m_v,
            b0, b1, b2, b3, obuf0, obuf1, sg0, sg1, ss0, ss1, acc_sh, hg_sh):
    cid = lax.axis_index("c")
    sid = lax.axis_index("s")
    wid = sid * NCORE + cid
    bufs = (b0, b1, b2, b3)
    obufs = (obuf0, obuf1)
    gsems = (sg0, sg1)
    ssems = (ss0, ss1)
    rs = pl.ds(sid * RPT, RPT)
    pltpu.sync_copy(hg_hbm.at[rs], hg_sh.at[rs])
    _zero_spmem(acc_sh, obuf0, sid, GF)
    plsc.subcore_barrier()
    pltpu.sync_copy(src_hbm.at[wid], src_v)
    pltpu.sync_copy(dst_hbm.at[wid], dst_v)
    pltpu.sync_copy(als_hbm, als_v)
    pltpu.sync_copy(ald_hbm, ald_v)
    pltpu.sync_copy(m_hbm, m_v)
    mvec = m_v[pl.ds(0, 16)]
    for ob in obufs:
        def zbody(r, carry):
            ob[r, pl.ds(32, 16)] = jnp.zeros((16,), _f32)
            return carry
        lax.fori_loop(0, EB, zbody, 0, unroll=4)

    for j in range(2):
        pltpu.async_copy(hg_sh.at[src_v.at[j]], bufs[j], gsems[j])

    def process(j, gbuf, ob):
        for g in range(EB // 16):
            sidx = src_v[j, pl.ds(g * 16, 16)]
            didx = dst_v[j, pl.ds(g * 16, 16)]
            el = plsc.load_gather(als_v, [sidx]) + plsc.load_gather(
                ald_v, [didx])
            e = jnp.where(el > 0, el, el * jnp.float32(0.2))
            p = jnp.exp(e - mvec)
            rows = lax.iota(_i32, 16) + g * 16
            cols = jnp.full((16,), 32, _i32)
            plsc.store_scatter(ob, [rows, cols], p)
            for t in range(16):
                r = g * 16 + t
                pb = p[jnp.full((16,), t, _i32)]
                ob[r, pl.ds(0, 16)] = gbuf[r, pl.ds(0, 16)] * pb
                ob[r, pl.ds(16, 16)] = gbuf[r, pl.ds(16, 16)] * pb

    def body(ii, carry):
        for q in range(4):
            j = ii * 4 + q
            bg = bufs[q]
            bn = bufs[(q + 2) % 4]
            ob = obufs[q % 2]
            @pl.when(j >= 2)
            def _():
                pltpu.make_async_copy(ob, acc_sh.at[dst_v.at[0]],
                                      ssems[q % 2]).wait()
            pltpu.make_async_copy(hg_sh.at[src_v.at[0]], bg,
                                  gsems[q % 2]).wait()
            @pl.when(j + 2 < NB)
            def _():
                pltpu.async_copy(hg_sh.at[src_v.at[j + 2]], bn,
                                 gsems[q % 2])
            process(j, bg, ob)
            pltpu.async_copy(ob, acc_sh.at[dst_v.at[j]], ssems[q % 2],
                             add=True)
        return carry
    lax.fori_loop(0, NB // 4, body, 0)
    for q in range(2):
        pltpu.make_async_copy(obufs[q], acc_sh.at[dst_v.at[0]],
                              ssems[q]).wait()
    plsc.subcore_barrier()
    _copy_out(acc_sh, out0, out1, cid, sid)


def _tc_call(body, out_shapes, *args):
    return pl.pallas_call(body, out_shape=out_shapes)(*args)


def _tcA1_body(x, w1, h1_o):
    h1_o[:, :] = jnp.dot(x[:, :], w1[:, :], preferred_element_type=_f32)


def _tcA2_body(d0, d1, h1, dinv_o, g1_o):
    deg = d0[:, 0:1] + d1[:, 0:1] + jnp.float32(1.0)
    dinv = jnp.where(deg > 0, lax.rsqrt(deg), jnp.float32(0.0))
    dinv_o[:, :] = dinv
    g1_o[:, :] = h1[:, :] * dinv


def _tcB_body(p0, p1, gp, dinv, b, w, g_o):
    dv = dinv[:, :]
    xk = jax.nn.relu(dv * (p0[:, :] + p1[:, :] + gp[:, :]) + b[:, :])
    g_o[:, :] = jnp.dot(xk, w[:, :], preferred_element_type=_f32) * dv


def _tcD_body(p0, p1, gp, dinv, b, wg, ats, atd, hg_o, als_o, ald_o, m_o):
    dv = dinv[:, :]
    x4 = jax.nn.relu(dv * (p0[:, :] + p1[:, :] + gp[:, :]) + b[:, :])
    hg = jnp.dot(x4, wg[:, :], preferred_element_type=_f32)
    hg_o[:, :] = hg
    als = jnp.sum(hg * ats[:, :], axis=1, keepdims=True)
    ald = jnp.sum(hg * atd[:, :], axis=1, keepdims=True)
    als_o[:, :] = als
    ald_o[:, :] = ald
    m = jax.nn.relu(jnp.max(als) + jnp.max(ald))
    m_o[:, :] = jnp.full((1, 16), m, _f32)


def _lstm_block(x, wt, bi, bh):
    g = jnp.dot(x, wt[:, :], preferred_element_type=_f32) + bi[:, :] + bh[:, :]
    i = g[:, 0:128]
    c = g[:, 256:384]
    o = g[:, 384:512]
    cs = jax.nn.sigmoid(i) * jnp.tanh(c)
    return jax.nn.sigmoid(o) * jnp.tanh(cs)


def _tcF_body(g0, g1, hg, als, ald, m16, bg, w0f, bi0f, bh0f, w0b, bi0b,
              bh0b, w1f, bi1f, bh1f, w1b, bi1b, bh1b,
              gam, bet, mu, var, wf, bf, y_o):
    el = als[:, :] + ald[:, :]
    e = jnp.where(el > 0, el, el * jnp.float32(0.2))
    p_self = jnp.exp(e - m16[:, 0:1])
    s = g0[:, 0:32] + g1[:, 0:32] + p_self * hg[:, :]
    den = g0[:, 32:33] + g1[:, 32:33] + p_self
    den = jnp.where(den > 0, den, jnp.float32(1.0))
    x5 = jax.nn.relu(s / den + bg[:, :])
    hf = _lstm_block(x5, w0f, bi0f, bh0f)
    hb = _lstm_block(x5, w0b, bi0b, bh0b)
    h1 = jnp.concatenate([hf, hb], axis=1)
    h1f = _lstm_block(h1, w1f, bi1f, bh1f)
    h1b = _lstm_block(h1, w1b, bi1b, bh1b)
    out = jnp.concatenate([h1f, h1b], axis=1)
    out = (out - mu[:, :]) * lax.rsqrt(var[:, :] + 1e-5) * gam[:, :] + bet[:, :]
    y_o[:, :] = jnp.dot(out, wf[:, :], preferred_element_type=_f32) + bf[:, :]


_BLK = 1024


def _tc_head(g0, g1, hg, als, ald, m16, bg, w0f, bi0f, bh0f, w0b, bi0b,
             bh0b, w1f, bi1f, bh1f, w1b, bi1b, bh1b, gam, bet, mu, var,
             wf, bf):
    rep2 = lambda shape: pl.BlockSpec(shape, lambda i: (0, 0))
    blk = lambda shape: pl.BlockSpec(shape, lambda i: (i, 0))
    return pl.pallas_call(
        _tcF_body,
        grid=(NP // _BLK,),
        in_specs=[
            blk((_BLK, GF)), blk((_BLK, GF)), blk((_BLK, 32)),
            blk((_BLK, 1)), blk((_BLK, 1)), rep2((1, 16)), rep2((1, 32)),
            rep2((32, 512)), rep2((1, 512)), rep2((1, 512)),
            rep2((32, 512)), rep2((1, 512)), rep2((1, 512)),
            rep2((256, 512)), rep2((1, 512)), rep2((1, 512)),
            rep2((256, 512)), rep2((1, 512)), rep2((1, 512)),
            rep2((1, 256)), rep2((1, 256)), rep2((1, 256)), rep2((1, 256)),
            rep2((256, 64)), rep2((1, 64)),
        ],
        out_specs=blk((_BLK, 64)),
        out_shape=jax.ShapeDtypeStruct((NP, 64), _f32),
    )(g0, g1, hg, als, ald, m16, bg, w0f, bi0f, bh0f, w0b, bi0b, bh0b,
      w1f, bi1f, bh1f, w1b, bi1b, bh1b, gam, bet, mu, var, wf, bf)


def kernel(x, edge_index, W1, b1, W2, b2, W3, b3, Wg, att_src, att_dst, bg,
           Wih0f, bih0f, bhh0f, Wih0b, bih0b, bhh0b,
           Wih1f, bih1f, bhh1f, Wih1b, bih1b, bhh1b,
           bn_gamma, bn_beta, bn_mean, bn_var, Wf, bf):
    pad = jnp.full((EP - E,), N, edge_index.dtype)
    src3 = jnp.concatenate([edge_index[0], pad]).reshape(NW, NB, EB)
    dst3 = jnp.concatenate([edge_index[1], pad]).reshape(NW, NB, EB)
    x_p = jnp.pad(x, ((0, NP - N), (0, 0)))

    r2 = lambda a: a.reshape(1, -1)

    d0, d1 = _sc_degree(dst3)
    H1 = _tc_call(_tcA1_body, jax.ShapeDtypeStruct((NP, 64), _f32), x_p, W1)
    dinv, G1 = _tc_call(
        _tcA2_body,
        (jax.ShapeDtypeStruct((NP, 1), _f32),
         jax.ShapeDtypeStruct((NP, 64), _f32)),
        d0, d1, H1)

    a0, a1 = _sc_agg64(src3, dst3, G1)
    G2 = _tc_call(_tcB_body, jax.ShapeDtypeStruct((NP, 32), _f32),
                  a0, a1, G1, dinv, r2(b1), W2)
    a0, a1 = _sc_agg32(src3, dst3, G2)
    G3 = _tc_call(_tcB_body, jax.ShapeDtypeStruct((NP, 32), _f32),
                  a0, a1, G2, dinv, r2(b2), W3)
    a0, a1 = _sc_agg32(src3, dst3, G3)
    Hg, als, ald, m16 = _tc_call(
        _tcD_body,
        (jax.ShapeDtypeStruct((NP, 32), _f32),
         jax.ShapeDtypeStruct((NP, 1), _f32),
         jax.ShapeDtypeStruct((NP, 1), _f32),
         jax.ShapeDtypeStruct((1, 16), _f32)),
        a0, a1, G3, dinv, r2(b3), Wg, r2(att_src), r2(att_dst))

    g0, g1 = _sc_gat(src3, dst3, Hg, als.reshape(NP), ald.reshape(NP),
                     m16.reshape(16))

    y = _tc_head(g0, g1, Hg, als, ald, m16, r2(bg),
                 Wih0f.T, r2(bih0f), r2(bhh0f),
                 Wih0b.T, r2(bih0b), r2(bhh0b),
                 Wih1f.T, r2(bih1f), r2(bhh1f),
                 Wih1b.T, r2(bih1b), r2(bhh1b),
                 r2(bn_gamma), r2(bn_beta), r2(bn_mean), r2(bn_var),
                 Wf, r2(bf))
    return y[:N]
